# baseline (speedup 1.0000x reference)
import functools

import jax
import jax.numpy as jnp
from jax import lax
from jax.experimental import pallas as pl
from jax.experimental.pallas import tpu as pltpu
from jax.experimental.pallas import tpu_sc as plsc

N_NODES = 100000
N_PAD = 102400
N_EDGES = 6400000
CHUNK = 256
CHUNKS = 784
NW = 32
E_PAD = NW * CHUNKS * CHUNK
MERGE_ROUNDS = 10
MERGE_RN = N_PAD // MERGE_ROUNDS
MERGE_WIN = MERGE_RN // 16

_mesh = plsc.VectorSubcoreMesh(core_axis_name="c", subcore_axis_name="s")


@functools.partial(
    pl.kernel,
    mesh=_mesh,
    compiler_params=pltpu.CompilerParams(needs_layout_passes=False),
    out_type=jax.ShapeDtypeStruct((2, N_PAD), jnp.float32),
    scratch_types=(
        [pltpu.VMEM((CHUNK,), jnp.int32)] * 4 +
        [pltpu.VMEM((CHUNK,), jnp.int32)] * 4 +
        [pltpu.VMEM((3 * CHUNK,), jnp.int32)] * 2 +
        [pltpu.VMEM((3 * CHUNK,), jnp.int32)] * 2 +
        [pltpu.VMEM((3 * CHUNK,), jnp.float32)] * 2 +
        [pltpu.VMEM((3 * CHUNK,), jnp.float32)] * 2 +
        [pltpu.VMEM((N_PAD,), jnp.float32),
         pltpu.VMEM((MERGE_WIN,), jnp.float32),
         pltpu.VMEM((MERGE_WIN,), jnp.float32),
         pltpu.VMEM_SHARED((16 * MERGE_RN,), jnp.float32),
         pltpu.SemaphoreType.DMA,
         pltpu.SemaphoreType.DMA]
    ),
)
def _lj_sc(posf_hbm, s_hbm, r_hbm, out_hbm, *scr):
    SIDX = scr[0:4]
    RIDX = scr[4:8]
    SFID = scr[8:10]
    RFID = scr[10:12]
    SCO = scr[12:14]
    RCO = scr[14:16]
    accum_v, accs_v, tmp_v, shared_v, isem, gsem = scr[16:22]
    cid = lax.axis_index("c")
    sid = lax.axis_index("s")
    wid = sid * 2 + cid
    wbase = wid * (CHUNKS * CHUNK)

    zero16 = jnp.zeros((16,), jnp.float32)
    lane = lax.iota(jnp.int32, 16)
    lane3 = lane * 3

    def zero_body(i, _):
        accum_v[pl.ds(i * 16, 16)] = zero16
        return _
    lax.fori_loop(0, N_PAD // 16, zero_body, None)

    def start_idx(t, s):
        base = wbase + t * CHUNK
        pltpu.async_copy(s_hbm.at[pl.ds(base, CHUNK)], SIDX[s], isem)
        pltpu.async_copy(r_hbm.at[pl.ds(base, CHUNK)], RIDX[s], isem)

    def wait_idx(s):
        pltpu.make_async_copy(s_hbm.at[pl.ds(0, CHUNK)], SIDX[s], isem).wait()
        pltpu.make_async_copy(r_hbm.at[pl.ds(0, CHUNK)], RIDX[s], isem).wait()

    def expand(s, b):
        def expand16(j, _):
            s16 = SIDX[s][pl.ds(j * 16, 16)]
            r16 = RIDX[s][pl.ds(j * 16, 16)]
            sf = s16 * 3
            rf = r16 * 3
            tgt = j * 48 + lane3
            plsc.store_scatter(SFID[b], [tgt], sf)
            plsc.store_scatter(SFID[b], [tgt + 1], sf + 1)
            plsc.store_scatter(SFID[b], [tgt + 2], sf + 2)
            plsc.store_scatter(RFID[b], [tgt], rf)
            plsc.store_scatter(RFID[b], [tgt + 1], rf + 1)
            plsc.store_scatter(RFID[b], [tgt + 2], rf + 2)
            return _
        lax.fori_loop(0, CHUNK // 16, expand16, None)

    def fire_gathers(b):
        pltpu.async_copy(posf_hbm.at[SFID[b]], SCO[b], gsem)
        pltpu.async_copy(posf_hbm.at[RFID[b]], RCO[b], gsem)

    def wait_gathers(b):
        pltpu.make_async_copy(posf_hbm.at[SFID[b]], SCO[b], gsem).wait()
        pltpu.make_async_copy(posf_hbm.at[RFID[b]], RCO[b], gsem).wait()

    def compute(s, b):
        def edge16(j, _):
            base3 = j * 48
            sx = plsc.load_gather(SCO[b], [base3 + lane3])
            sy = plsc.load_gather(SCO[b], [base3 + lane3 + 1])
            sz = plsc.load_gather(SCO[b], [base3 + lane3 + 2])
            rx = plsc.load_gather(RCO[b], [base3 + lane3])
            ry = plsc.load_gather(RCO[b], [base3 + lane3 + 1])
            rz = plsc.load_gather(RCO[b], [base3 + lane3 + 2])
            dx = rx - sx
            dy = ry - sy
            dz = rz - sz
            r2 = dx * dx + dy * dy + dz * dz
            inv = 1.0 / r2
            e = inv * inv * inv
            en = 2.0 * (e * e - e)
            ridx16 = RIDX[s][pl.ds(j * 16, 16)]
            plsc.addupdate_scatter(accum_v, [ridx16], en)
            return _
        lax.fori_loop(0, CHUNK // 16, edge16, None)

    def step(t, k, start_t3):
        wait_idx((k + 1) % 4)
        expand((k + 1) % 4, (k + 1) % 2)
        fire_gathers((k + 1) % 2)
        wait_gathers(k % 2)
        compute(k % 4, k % 2)
        if start_t3:
            start_idx(t + 3, (k + 3) % 4)

    start_idx(0, 0)
    start_idx(1, 1)
    start_idx(2, 2)
    wait_idx(0)
    expand(0, 0)
    fire_gathers(0)

    def pipe_body(u4, _):
        t0 = u4 * 4
        step(t0 + 0, 0, True)
        step(t0 + 1, 1, True)
        step(t0 + 2, 2, True)
        step(t0 + 3, 3, True)
        return _

    lax.fori_loop(0, (CHUNKS - 4) // 4, pipe_body, None)

    step(CHUNKS - 4, 0, True)
    step(CHUNKS - 3, 1, False)
    step(CHUNKS - 2, 2, False)
    wait_gathers(1)
    compute(3, 1)

    def add_body(i, _):
        plsc.addupdate(accs_v.at[pl.ds(i * 16, 16)], tmp_v[pl.ds(i * 16, 16)])
        return _

    off = sid * MERGE_WIN
    for rr in range(MERGE_ROUNDS):
        pltpu.sync_copy(accum_v.at[pl.ds(rr * MERGE_RN, MERGE_RN)],
                        shared_v.at[pl.ds(sid * MERGE_RN, MERGE_RN)])
        plsc.subcore_barrier()
        pltpu.sync_copy(shared_v.at[pl.ds(off, MERGE_WIN)], accs_v)
        for src in range(1, 16):
            pltpu.sync_copy(
                shared_v.at[pl.ds(src * MERGE_RN + off, MERGE_WIN)], tmp_v)
            lax.fori_loop(0, MERGE_WIN // 16, add_body, None)
        pltpu.sync_copy(accs_v, out_hbm.at[cid, pl.ds(rr * MERGE_RN + off, MERGE_WIN)])
        plsc.subcore_barrier()


def _tc_sum_body(a_ref, o_ref):
    o_ref[...] = a_ref[0] + a_ref[1]


_tc_sum = pl.pallas_call(
    _tc_sum_body,
    out_shape=jax.ShapeDtypeStruct((N_PAD // 128, 128), jnp.float32),
)


def kernel(positions, senders, receivers):
    pos_flat = jnp.concatenate(
        [positions.reshape(-1), jnp.zeros((3 * (N_PAD - N_NODES),), jnp.float32)])
    pad = E_PAD - N_EDGES
    s_pad = jnp.concatenate([senders, jnp.zeros((pad,), jnp.int32)])
    r_pad = jnp.concatenate([receivers, jnp.full((pad,), N_NODES, jnp.int32)])
    partials = _lj_sc(pos_flat, s_pad, r_pad)
    summed = _tc_sum(partials.reshape(2, N_PAD // 128, 128))
    return summed.reshape(-1)[:N_NODES]


# speedup vs baseline: 4.8334x; 4.8334x over previous
import jax
import jax.numpy as jnp
from jax import lax
from jax.experimental import pallas as pl
from jax.experimental.pallas import tpu as pltpu
from jax.experimental.pallas import tpu_sc as plsc

N_NODES = 100000
N_PAD = 102400
N_EDGES = 6400000
CHUNK = 1024
CHUNKS = 196
NW = 32
E_PAD = NW * CHUNKS * CHUNK
MERGE_ROUNDS = 10
MERGE_RN = N_PAD // MERGE_ROUNDS
MERGE_WIN = MERGE_RN // 16

_mesh = plsc.VectorSubcoreMesh(core_axis_name="c", subcore_axis_name="s")


@pl.kernel(
    mesh=_mesh,
    compiler_params=pltpu.CompilerParams(needs_layout_passes=False),
    out_type=(jax.ShapeDtypeStruct((2, N_PAD), jnp.float32),
              jax.ShapeDtypeStruct((E_PAD,), jnp.float32)),
    scratch_types=(
        [pltpu.VMEM((CHUNK,), jnp.int32)] * 4 +
        [pltpu.VMEM((CHUNK,), jnp.int32)] * 4 +
        [pltpu.VMEM((CHUNK,), jnp.float32)] * 4 +
        [pltpu.VMEM((CHUNK,), jnp.float32)] * 2 +
        [pltpu.VMEM((N_PAD,), jnp.float32),
         pltpu.VMEM((MERGE_WIN,), jnp.float32),
         pltpu.VMEM((MERGE_WIN,), jnp.float32),
         pltpu.VMEM_SHARED((16 * MERGE_RN,), jnp.float32),
         pltpu.SemaphoreType.DMA,
         pltpu.SemaphoreType.DMA,
         pltpu.SemaphoreType.DMA]
    ),
)
def _lj_sc(px_hbm, py_hbm, pz_hbm, s_hbm, r_hbm, out_hbm, r2_hbm, *scr):
    SIDX = scr[0:4]
    RIDX = scr[4:8]
    R2IN = scr[8:12]
    R2OUT = scr[12:14]
    big_v, accs_v, tmp_v, shared_v, isem, rsem, wsem = scr[14:21]
    cid = lax.axis_index("c")
    sid = lax.axis_index("s")
    wid = sid * 2 + cid
    wbase = wid * (CHUNKS * CHUNK)

    zero16 = jnp.zeros((16,), jnp.float32)

    def start_idx(t, k, with_sender):
        base = wbase + t * CHUNK
        if with_sender:
            pltpu.async_copy(s_hbm.at[pl.ds(base, CHUNK)], SIDX[k], isem)
        pltpu.async_copy(r_hbm.at[pl.ds(base, CHUNK)], RIDX[k], isem)

    def wait_idx(k, with_sender):
        if with_sender:
            pltpu.make_async_copy(s_hbm.at[pl.ds(0, CHUNK)], SIDX[k], isem).wait()
        pltpu.make_async_copy(r_hbm.at[pl.ds(0, CHUNK)], RIDX[k], isem).wait()

    def start_r2in(t, k):
        base = wbase + t * CHUNK
        pltpu.async_copy(r2_hbm.at[pl.ds(base, CHUNK)], R2IN[k], rsem)

    def wait_r2in(k):
        pltpu.make_async_copy(r2_hbm.at[pl.ds(0, CHUNK)], R2IN[k], rsem).wait()

    def start_wb(t, b):
        base = wbase + t * CHUNK
        pltpu.async_copy(R2OUT[b], r2_hbm.at[pl.ds(base, CHUNK)], wsem)

    def wait_wb(b):
        pltpu.make_async_copy(R2OUT[b], r2_hbm.at[pl.ds(0, CHUNK)], wsem).wait()

    def compute_pass(p, k, b):
        @plsc.parallel_loop(0, CHUNK // 16)
        def _(j):
            o = pl.ds(j * 16, 16)
            s16 = SIDX[k][o]
            r16 = RIDX[k][o]
            sv = plsc.load_gather(big_v, [s16])
            rv = plsc.load_gather(big_v, [r16])
            d = rv - sv
            d2 = d * d
            if p == 0:
                R2OUT[b][o] = d2
            elif p == 1:
                R2OUT[b][o] = R2IN[k][o] + d2
            else:
                r2 = R2IN[k][o] + d2
                inv = 1.0 / r2
                e = inv * inv * inv
                R2OUT[b][o] = 2.0 * (e * e - e)

    def pass_step(p, t, k, wb_guard, prefetch):
        wait_idx(k, True)
        if p > 0:
            wait_r2in(k)
        if wb_guard:
            wait_wb(k % 2)
        compute_pass(p, k, k % 2)
        start_wb(t, k % 2)
        if prefetch:
            start_idx(t + 4, k, True)
            if p > 0:
                start_r2in(t + 4, k)

    for p in range(3):
        tab = (px_hbm, py_hbm, pz_hbm)[p]
        pltpu.sync_copy(tab, big_v)
        for t0 in range(4):
            start_idx(t0, t0, True)
            if p > 0:
                start_r2in(t0, t0)
        pass_step(p, 0, 0, False, True)
        pass_step(p, 1, 1, False, True)
        pass_step(p, 2, 2, True, True)
        pass_step(p, 3, 3, True, True)

        def pass_body(u4, _, p=p):
            t0 = u4 * 4 + 4
            pass_step(p, t0 + 0, 0, True, True)
            pass_step(p, t0 + 1, 1, True, True)
            pass_step(p, t0 + 2, 2, True, True)
            pass_step(p, t0 + 3, 3, True, True)
            return _

        lax.fori_loop(0, (CHUNKS - 8) // 4, pass_body, None)
        pass_step(p, CHUNKS - 4, 0, True, False)
        pass_step(p, CHUNKS - 3, 1, True, False)
        pass_step(p, CHUNKS - 2, 2, True, False)
        pass_step(p, CHUNKS - 1, 3, True, False)
        wait_wb(0)
        wait_wb(1)

    def zero_body(i, _):
        big_v[pl.ds(i * 16, 16)] = zero16
        return _
    lax.fori_loop(0, N_PAD // 16, zero_body, None)

    for t0 in range(4):
        start_idx(t0, t0, False)
        start_r2in(t0, t0)

    def scat_step(t, k, prefetch):
        wait_idx(k, False)
        wait_r2in(k)

        def edge16(j, _):
            o = pl.ds(j * 16, 16)
            plsc.addupdate_scatter(big_v, [RIDX[k][o]], R2IN[k][o])
            return _
        lax.fori_loop(0, CHUNK // 16, edge16, None)
        if prefetch:
            start_idx(t + 4, k, False)
            start_r2in(t + 4, k)

    def scat_body(u4, _):
        t0 = u4 * 4
        scat_step(t0 + 0, 0, True)
        scat_step(t0 + 1, 1, True)
        scat_step(t0 + 2, 2, True)
        scat_step(t0 + 3, 3, True)
        return _

    lax.fori_loop(0, (CHUNKS - 4) // 4, scat_body, None)
    scat_step(CHUNKS - 4, 0, False)
    scat_step(CHUNKS - 3, 1, False)
    scat_step(CHUNKS - 2, 2, False)
    scat_step(CHUNKS - 1, 3, False)

    def add_body(i, _):
        plsc.addupdate(accs_v.at[pl.ds(i * 16, 16)], tmp_v[pl.ds(i * 16, 16)])
        return _

    off = sid * MERGE_WIN
    for rr in range(MERGE_ROUNDS):
        pltpu.sync_copy(big_v.at[pl.ds(rr * MERGE_RN, MERGE_RN)],
                        shared_v.at[pl.ds(sid * MERGE_RN, MERGE_RN)])
        plsc.subcore_barrier()
        pltpu.sync_copy(shared_v.at[pl.ds(off, MERGE_WIN)], accs_v)
        for src in range(1, 16):
            pltpu.sync_copy(
                shared_v.at[pl.ds(src * MERGE_RN + off, MERGE_WIN)], tmp_v)
            lax.fori_loop(0, MERGE_WIN // 16, add_body, None)
        pltpu.sync_copy(accs_v, out_hbm.at[cid, pl.ds(rr * MERGE_RN + off, MERGE_WIN)])
        plsc.subcore_barrier()


def _tc_sum_body(a_ref, o_ref):
    o_ref[...] = a_ref[0] + a_ref[1]


_tc_sum = pl.pallas_call(
    _tc_sum_body,
    out_shape=jax.ShapeDtypeStruct((N_PAD // 128, 128), jnp.float32),
)


def kernel(positions, senders, receivers):
    pos_t = jnp.concatenate(
        [positions.T, jnp.zeros((3, N_PAD - N_NODES), jnp.float32)], axis=1)
    px, py, pz = pos_t[0], pos_t[1], pos_t[2]
    pad = E_PAD - N_EDGES
    s_pad = jnp.concatenate([senders, jnp.zeros((pad,), jnp.int32)])
    r_pad = jnp.concatenate([receivers, jnp.full((pad,), N_NODES, jnp.int32)])
    partials, _ = _lj_sc(px, py, pz, s_pad, r_pad)
    summed = _tc_sum(partials.reshape(2, N_PAD // 128, 128))
    return summed.reshape(-1)[:N_NODES]


# speedup vs baseline: 6.3993x; 1.3240x over previous
import jax
import jax.numpy as jnp
from jax import lax
from jax.experimental import pallas as pl
from jax.experimental.pallas import tpu as pltpu
from jax.experimental.pallas import tpu_sc as plsc

N_NODES = 100000
N_PAD = 102400
N_EDGES = 6400000
CHUNK = 1024
CHUNKS = 196
NW = 32
E_PAD = NW * CHUNKS * CHUNK

_mesh = plsc.VectorSubcoreMesh(core_axis_name="c", subcore_axis_name="s")


@pl.kernel(
    mesh=_mesh,
    compiler_params=pltpu.CompilerParams(needs_layout_passes=False),
    out_type=(jax.ShapeDtypeStruct((NW * N_PAD,), jnp.float32),
              jax.ShapeDtypeStruct((E_PAD,), jnp.float32)),
    scratch_types=(
        [pltpu.VMEM((CHUNK,), jnp.int32)] * 4 +
        [pltpu.VMEM((CHUNK,), jnp.int32)] * 4 +
        [pltpu.VMEM((CHUNK,), jnp.float32)] * 4 +
        [pltpu.VMEM((CHUNK,), jnp.float32)] * 2 +
        [pltpu.VMEM((N_PAD,), jnp.float32),
         pltpu.SemaphoreType.DMA,
         pltpu.SemaphoreType.DMA,
         pltpu.SemaphoreType.DMA]
    ),
)
def _lj_sc(px_hbm, py_hbm, pz_hbm, zeros_hbm, s_hbm, r_hbm, out_hbm, r2_hbm,
           *scr):
    SIDX = scr[0:4]
    RIDX = scr[4:8]
    R2IN = scr[8:12]
    R2OUT = scr[12:14]
    big_v, isem, rsem, wsem = scr[14:18]
    cid = lax.axis_index("c")
    sid = lax.axis_index("s")
    wid = sid * 2 + cid
    wbase = wid * (CHUNKS * CHUNK)

    def start_idx(t, k, with_sender):
        base = wbase + t * CHUNK
        if with_sender:
            pltpu.async_copy(s_hbm.at[pl.ds(base, CHUNK)], SIDX[k], isem)
        pltpu.async_copy(r_hbm.at[pl.ds(base, CHUNK)], RIDX[k], isem)

    def wait_idx(k, with_sender):
        if with_sender:
            pltpu.make_async_copy(s_hbm.at[pl.ds(0, CHUNK)], SIDX[k], isem).wait()
        pltpu.make_async_copy(r_hbm.at[pl.ds(0, CHUNK)], RIDX[k], isem).wait()

    def start_r2in(t, k):
        base = wbase + t * CHUNK
        pltpu.async_copy(r2_hbm.at[pl.ds(base, CHUNK)], R2IN[k], rsem)

    def wait_r2in(k):
        pltpu.make_async_copy(r2_hbm.at[pl.ds(0, CHUNK)], R2IN[k], rsem).wait()

    def start_wb(t, b):
        base = wbase + t * CHUNK
        pltpu.async_copy(R2OUT[b], r2_hbm.at[pl.ds(base, CHUNK)], wsem)

    def wait_wb(b):
        pltpu.make_async_copy(R2OUT[b], r2_hbm.at[pl.ds(0, CHUNK)], wsem).wait()

    def compute_pass(p, k, b):
        @plsc.parallel_loop(0, CHUNK // 16, unroll=4)
        def _(j):
            o = pl.ds(j * 16, 16)
            s16 = SIDX[k][o]
            r16 = RIDX[k][o]
            sv = plsc.load_gather(big_v, [s16])
            rv = plsc.load_gather(big_v, [r16])
            d = rv - sv
            d2 = d * d
            if p == 0:
                R2OUT[b][o] = d2
            elif p == 1:
                R2OUT[b][o] = R2IN[k][o] + d2
            else:
                r2 = R2IN[k][o] + d2
                inv = 1.0 / r2
                e = inv * inv * inv
                R2OUT[b][o] = 2.0 * (e * e - e)

    def pass_step(p, t, k, wb_guard, prefetch):
        wait_idx(k, True)
        if p > 0:
            wait_r2in(k)
        if wb_guard:
            wait_wb(k % 2)
        compute_pass(p, k, k % 2)
        start_wb(t, k % 2)
        if prefetch:
            start_idx(t + 4, k, True)
            if p > 0:
                start_r2in(t + 4, k)

    for p in range(3):
        tab = (px_hbm, py_hbm, pz_hbm)[p]
        pltpu.sync_copy(tab, big_v)
        for t0 in range(4):
            start_idx(t0, t0, True)
            if p > 0:
                start_r2in(t0, t0)
        pass_step(p, 0, 0, False, True)
        pass_step(p, 1, 1, False, True)
        pass_step(p, 2, 2, True, True)
        pass_step(p, 3, 3, True, True)

        def pass_body(u4, _, p=p):
            t0 = u4 * 4 + 4
            pass_step(p, t0 + 0, 0, True, True)
            pass_step(p, t0 + 1, 1, True, True)
            pass_step(p, t0 + 2, 2, True, True)
            pass_step(p, t0 + 3, 3, True, True)
            return _

        lax.fori_loop(0, (CHUNKS - 8) // 4, pass_body, None)
        pass_step(p, CHUNKS - 4, 0, True, False)
        pass_step(p, CHUNKS - 3, 1, True, False)
        pass_step(p, CHUNKS - 2, 2, True, False)
        pass_step(p, CHUNKS - 1, 3, True, False)
        wait_wb(0)
        wait_wb(1)

    pltpu.sync_copy(zeros_hbm, big_v)

    for t0 in range(4):
        start_idx(t0, t0, False)
        start_r2in(t0, t0)

    def scat_step(t, k, prefetch):
        wait_idx(k, False)
        wait_r2in(k)

        def edge16(j, _):
            o = pl.ds(j * 16, 16)
            plsc.addupdate_scatter(big_v, [RIDX[k][o]], R2IN[k][o])
            return _
        lax.fori_loop(0, CHUNK // 16, edge16, None, unroll=4)
        if prefetch:
            start_idx(t + 4, k, False)
            start_r2in(t + 4, k)

    def scat_body(u4, _):
        t0 = u4 * 4
        scat_step(t0 + 0, 0, True)
        scat_step(t0 + 1, 1, True)
        scat_step(t0 + 2, 2, True)
        scat_step(t0 + 3, 3, True)
        return _

    lax.fori_loop(0, (CHUNKS - 4) // 4, scat_body, None)
    scat_step(CHUNKS - 4, 0, False)
    scat_step(CHUNKS - 3, 1, False)
    scat_step(CHUNKS - 2, 2, False)
    scat_step(CHUNKS - 1, 3, False)

    pltpu.sync_copy(big_v, out_hbm.at[pl.ds(wid * N_PAD, N_PAD)])


def _tc_sum_body(a_ref, o_ref):
    o_ref[...] = jnp.sum(a_ref[...], axis=0)


_tc_sum = pl.pallas_call(
    _tc_sum_body,
    out_shape=jax.ShapeDtypeStruct((N_PAD // 128, 128), jnp.float32),
)


def kernel(positions, senders, receivers):
    pos_t = jnp.concatenate(
        [positions.T, jnp.zeros((3, N_PAD - N_NODES), jnp.float32)], axis=1)
    px, py, pz = pos_t[0], pos_t[1], pos_t[2]
    zeros = jnp.zeros((N_PAD,), jnp.float32)
    pad = E_PAD - N_EDGES
    s_pad = jnp.concatenate([senders, jnp.zeros((pad,), jnp.int32)])
    r_pad = jnp.concatenate([receivers, jnp.full((pad,), N_NODES, jnp.int32)])
    partials, _ = _lj_sc(px, py, pz, zeros, s_pad, r_pad)
    summed = _tc_sum(partials.reshape(NW, N_PAD // 128, 128))
    return summed.reshape(-1)[:N_NODES]


# speedup vs baseline: 7.8181x; 1.2217x over previous
import jax
import jax.numpy as jnp
from jax import lax
from jax.experimental import pallas as pl
from jax.experimental.pallas import tpu as pltpu
from jax.experimental.pallas import tpu_sc as plsc

N_NODES = 100000
N_PAD = 100096
N_EDGES = 6400000
CHUNK = 2000
CHUNKS = 100
NW = 32

_mesh = plsc.VectorSubcoreMesh(core_axis_name="c", subcore_axis_name="s")


@pl.kernel(
    mesh=_mesh,
    compiler_params=pltpu.CompilerParams(needs_layout_passes=False),
    out_type=(jax.ShapeDtypeStruct((NW * N_PAD,), jnp.float32),
              jax.ShapeDtypeStruct((N_EDGES,), jnp.float32)),
    scratch_types=(
        [pltpu.VMEM((CHUNK,), jnp.int32)] * 4 +
        [pltpu.VMEM((CHUNK,), jnp.int32)] * 4 +
        [pltpu.VMEM((CHUNK,), jnp.float32)] * 4 +
        [pltpu.VMEM((CHUNK,), jnp.float32)] * 2 +
        [pltpu.VMEM((N_PAD,), jnp.float32),
         pltpu.SemaphoreType.DMA,
         pltpu.SemaphoreType.DMA,
         pltpu.SemaphoreType.DMA]
    ),
)
def _lj_sc(px_hbm, py_hbm, pz_hbm, zeros_hbm, s_hbm, r_hbm, out_hbm, r2_hbm,
           *scr):
    SIDX = scr[0:4]
    RIDX = scr[4:8]
    R2IN = scr[8:12]
    R2OUT = scr[12:14]
    big_v, isem, rsem, wsem = scr[14:18]
    cid = lax.axis_index("c")
    sid = lax.axis_index("s")
    wid = sid * 2 + cid
    wbase = wid * (CHUNKS * CHUNK)

    def start_idx(t, k, with_sender):
        base = wbase + t * CHUNK
        if with_sender:
            pltpu.async_copy(s_hbm.at[pl.ds(base, CHUNK)], SIDX[k], isem)
        pltpu.async_copy(r_hbm.at[pl.ds(base, CHUNK)], RIDX[k], isem)

    def wait_idx(k, with_sender):
        if with_sender:
            pltpu.make_async_copy(s_hbm.at[pl.ds(0, CHUNK)], SIDX[k], isem).wait()
        pltpu.make_async_copy(r_hbm.at[pl.ds(0, CHUNK)], RIDX[k], isem).wait()

    def start_r2in(t, k):
        base = wbase + t * CHUNK
        pltpu.async_copy(r2_hbm.at[pl.ds(base, CHUNK)], R2IN[k], rsem)

    def wait_r2in(k):
        pltpu.make_async_copy(r2_hbm.at[pl.ds(0, CHUNK)], R2IN[k], rsem).wait()

    def start_wb(t, b):
        base = wbase + t * CHUNK
        pltpu.async_copy(R2OUT[b], r2_hbm.at[pl.ds(base, CHUNK)], wsem)

    def wait_wb(b):
        pltpu.make_async_copy(R2OUT[b], r2_hbm.at[pl.ds(0, CHUNK)], wsem).wait()

    def compute_pass(p, k, b):
        @plsc.parallel_loop(0, CHUNK // 16, unroll=5)
        def _(j):
            o = pl.ds(j * 16, 16)
            s16 = SIDX[k][o]
            r16 = RIDX[k][o]
            sv = plsc.load_gather(big_v, [s16])
            rv = plsc.load_gather(big_v, [r16])
            d = rv - sv
            d2 = d * d
            if p == 0:
                R2OUT[b][o] = d2
            elif p == 1:
                R2OUT[b][o] = R2IN[k][o] + d2
            else:
                r2 = R2IN[k][o] + d2
                inv = 1.0 / r2
                e = inv * inv * inv
                R2OUT[b][o] = 2.0 * (e * e - e)

    def pass_step(p, t, k, wb_guard, prefetch):
        wait_idx(k, True)
        if p > 0:
            wait_r2in(k)
        if wb_guard:
            wait_wb(k % 2)
        compute_pass(p, k, k % 2)
        start_wb(t, k % 2)
        if prefetch:
            start_idx(t + 4, k, True)
            if p > 0:
                start_r2in(t + 4, k)

    for p in range(3):
        tab = (px_hbm, py_hbm, pz_hbm)[p]
        pltpu.sync_copy(tab, big_v)
        for t0 in range(4):
            start_idx(t0, t0, True)
            if p > 0:
                start_r2in(t0, t0)
        pass_step(p, 0, 0, False, True)
        pass_step(p, 1, 1, False, True)
        pass_step(p, 2, 2, True, True)
        pass_step(p, 3, 3, True, True)

        def pass_body(u4, _, p=p):
            t0 = u4 * 4 + 4
            pass_step(p, t0 + 0, 0, True, True)
            pass_step(p, t0 + 1, 1, True, True)
            pass_step(p, t0 + 2, 2, True, True)
            pass_step(p, t0 + 3, 3, True, True)
            return _

        lax.fori_loop(0, (CHUNKS - 8) // 4, pass_body, None)
        pass_step(p, CHUNKS - 4, 0, True, False)
        pass_step(p, CHUNKS - 3, 1, True, False)
        pass_step(p, CHUNKS - 2, 2, True, False)
        pass_step(p, CHUNKS - 1, 3, True, False)
        wait_wb(0)
        wait_wb(1)

    pltpu.sync_copy(zeros_hbm, big_v)

    for t0 in range(4):
        start_idx(t0, t0, False)
        start_r2in(t0, t0)

    def scat_step(t, k, prefetch):
        wait_idx(k, False)
        wait_r2in(k)

        def edge16(j, _):
            o = pl.ds(j * 16, 16)
            plsc.addupdate_scatter(big_v, [RIDX[k][o]], R2IN[k][o])
            return _
        lax.fori_loop(0, CHUNK // 16, edge16, None, unroll=5)
        if prefetch:
            start_idx(t + 4, k, False)
            start_r2in(t + 4, k)

    def scat_body(u4, _):
        t0 = u4 * 4
        scat_step(t0 + 0, 0, True)
        scat_step(t0 + 1, 1, True)
        scat_step(t0 + 2, 2, True)
        scat_step(t0 + 3, 3, True)
        return _

    lax.fori_loop(0, (CHUNKS - 4) // 4, scat_body, None)
    scat_step(CHUNKS - 4, 0, False)
    scat_step(CHUNKS - 3, 1, False)
    scat_step(CHUNKS - 2, 2, False)
    scat_step(CHUNKS - 1, 3, False)

    pltpu.sync_copy(big_v, out_hbm.at[pl.ds(wid * N_PAD, N_PAD)])


def _tc_sum_body(a_ref, o_ref):
    o_ref[...] = jnp.sum(a_ref[...], axis=0)


_tc_sum = pl.pallas_call(
    _tc_sum_body,
    out_shape=jax.ShapeDtypeStruct((N_PAD // 128, 128), jnp.float32),
)


def kernel(positions, senders, receivers):
    pos_t = jnp.concatenate(
        [positions.T, jnp.zeros((3, N_PAD - N_NODES), jnp.float32)], axis=1)
    px, py, pz = pos_t[0], pos_t[1], pos_t[2]
    zeros = jnp.zeros((N_PAD,), jnp.float32)
    partials, _ = _lj_sc(px, py, pz, zeros, senders, receivers)
    summed = _tc_sum(partials.reshape(NW, N_PAD // 128, 128))
    return summed.reshape(-1)[:N_NODES]


# speedup vs baseline: 7.9318x; 1.0145x over previous
import jax
import jax.numpy as jnp
from jax import lax
from jax.experimental import pallas as pl
from jax.experimental.pallas import tpu as pltpu
from jax.experimental.pallas import tpu_sc as plsc

N_NODES = 100000
N_PAD = 100096
N_EDGES = 6400000
CHUNK = 2000
CHUNKS = 100
NW = 32

_mesh = plsc.VectorSubcoreMesh(core_axis_name="c", subcore_axis_name="s")


@pl.kernel(
    mesh=_mesh,
    compiler_params=pltpu.CompilerParams(needs_layout_passes=False),
    out_type=(jax.ShapeDtypeStruct((NW * N_PAD,), jnp.float32),
              jax.ShapeDtypeStruct((N_EDGES,), jnp.float32)),
    scratch_types=(
        [pltpu.VMEM((CHUNK,), jnp.int32)] * 4 +
        [pltpu.VMEM((CHUNK,), jnp.int32)] * 4 +
        [pltpu.VMEM((CHUNK,), jnp.float32)] * 4 +
        [pltpu.VMEM((CHUNK,), jnp.float32)] * 2 +
        [pltpu.VMEM((N_PAD,), jnp.float32),
         pltpu.SemaphoreType.DMA,
         pltpu.SemaphoreType.DMA,
         pltpu.SemaphoreType.DMA]
    ),
)
def _lj_sc(px_hbm, py_hbm, pz_hbm, zeros_hbm, s_hbm, r_hbm, out_hbm, r2_hbm,
           *scr):
    SIDX = scr[0:4]
    RIDX = scr[4:8]
    R2IN = scr[8:12]
    R2OUT = scr[12:14]
    big_v, isem, rsem, wsem = scr[14:18]
    cid = lax.axis_index("c")
    sid = lax.axis_index("s")
    wid = sid * 2 + cid
    wbase = wid * (CHUNKS * CHUNK)

    def start_idx(t, k, with_sender):
        base = wbase + t * CHUNK
        if with_sender:
            pltpu.async_copy(s_hbm.at[pl.ds(base, CHUNK)], SIDX[k], isem)
        pltpu.async_copy(r_hbm.at[pl.ds(base, CHUNK)], RIDX[k], isem)

    def wait_idx(k, with_sender):
        if with_sender:
            pltpu.make_async_copy(s_hbm.at[pl.ds(0, CHUNK)], SIDX[k], isem).wait()
        pltpu.make_async_copy(r_hbm.at[pl.ds(0, CHUNK)], RIDX[k], isem).wait()

    def start_r2in(t, k):
        base = wbase + t * CHUNK
        pltpu.async_copy(r2_hbm.at[pl.ds(base, CHUNK)], R2IN[k], rsem)

    def wait_r2in(k):
        pltpu.make_async_copy(r2_hbm.at[pl.ds(0, CHUNK)], R2IN[k], rsem).wait()

    def start_wb(t, b):
        base = wbase + t * CHUNK
        pltpu.async_copy(R2OUT[b], r2_hbm.at[pl.ds(base, CHUNK)], wsem)

    def wait_wb(b):
        pltpu.make_async_copy(R2OUT[b], r2_hbm.at[pl.ds(0, CHUNK)], wsem).wait()

    def compute_pass(p, k, b):
        @plsc.parallel_loop(0, CHUNK // 16, unroll=5)
        def _(j):
            o = pl.ds(j * 16, 16)
            s16 = SIDX[k][o]
            r16 = RIDX[k][o]
            sv = plsc.load_gather(big_v, [s16])
            rv = plsc.load_gather(big_v, [r16])
            d = rv - sv
            d2 = d * d
            if p == 0:
                R2OUT[b][o] = d2
            elif p == 1:
                R2OUT[b][o] = R2IN[k][o] + d2
            else:
                r2 = R2IN[k][o] + d2
                inv = 1.0 / r2
                e = inv * inv * inv
                R2OUT[b][o] = 2.0 * (e * e - e)

    def pass_step(p, t, k, wb_guard, prefetch):
        wait_idx(k, True)
        if p > 0:
            wait_r2in(k)
        if wb_guard:
            wait_wb(k % 2)
        compute_pass(p, k, k % 2)
        start_wb(t, k % 2)
        if prefetch:
            start_idx(t + 4, k, True)
            if p > 0:
                start_r2in(t + 4, k)

    for p in range(3):
        for t0 in range(4):
            start_idx(t0, t0, True)
            if p > 0:
                start_r2in(t0, t0)
        tab = (px_hbm, py_hbm, pz_hbm)[p]
        pltpu.sync_copy(tab, big_v)
        pass_step(p, 0, 0, False, True)
        pass_step(p, 1, 1, False, True)
        pass_step(p, 2, 2, True, True)
        pass_step(p, 3, 3, True, True)

        def pass_body(u4, _, p=p):
            t0 = u4 * 4 + 4
            pass_step(p, t0 + 0, 0, True, True)
            pass_step(p, t0 + 1, 1, True, True)
            pass_step(p, t0 + 2, 2, True, True)
            pass_step(p, t0 + 3, 3, True, True)
            return _

        lax.fori_loop(0, (CHUNKS - 8) // 4, pass_body, None)
        pass_step(p, CHUNKS - 4, 0, True, False)
        pass_step(p, CHUNKS - 3, 1, True, False)
        pass_step(p, CHUNKS - 2, 2, True, False)
        pass_step(p, CHUNKS - 1, 3, True, False)
        wait_wb(0)
        wait_wb(1)

    pltpu.sync_copy(zeros_hbm, big_v)

    for t0 in range(4):
        start_idx(t0, t0, False)
        start_r2in(t0, t0)

    def scat_step(t, k, prefetch):
        wait_idx(k, False)
        wait_r2in(k)

        def edge16(j, _):
            o = pl.ds(j * 16, 16)
            plsc.addupdate_scatter(big_v, [RIDX[k][o]], R2IN[k][o])
            return _
        lax.fori_loop(0, CHUNK // 16, edge16, None, unroll=5)
        if prefetch:
            start_idx(t + 4, k, False)
            start_r2in(t + 4, k)

    def scat_body(u4, _):
        t0 = u4 * 4
        scat_step(t0 + 0, 0, True)
        scat_step(t0 + 1, 1, True)
        scat_step(t0 + 2, 2, True)
        scat_step(t0 + 3, 3, True)
        return _

    lax.fori_loop(0, (CHUNKS - 4) // 4, scat_body, None)
    scat_step(CHUNKS - 4, 0, False)
    scat_step(CHUNKS - 3, 1, False)
    scat_step(CHUNKS - 2, 2, False)
    scat_step(CHUNKS - 1, 3, False)

    pltpu.sync_copy(big_v, out_hbm.at[pl.ds(wid * N_PAD, N_PAD)])


def _tc_sum_body(a_ref, o_ref):
    o_ref[...] = jnp.sum(a_ref[...], axis=0)


_tc_sum = pl.pallas_call(
    _tc_sum_body,
    out_shape=jax.ShapeDtypeStruct((N_PAD // 128, 128), jnp.float32),
)


def kernel(positions, senders, receivers):
    pos_t = jnp.concatenate(
        [positions.T, jnp.zeros((3, N_PAD - N_NODES), jnp.float32)], axis=1)
    px, py, pz = pos_t[0], pos_t[1], pos_t[2]
    zeros = jnp.zeros((N_PAD,), jnp.float32)
    partials, _ = _lj_sc(px, py, pz, zeros, senders, receivers)
    summed = _tc_sum(partials.reshape(NW, N_PAD // 128, 128))
    return summed.reshape(-1)[:N_NODES]
